# baseline (speedup 1.0000x reference)
import jax
import jax.numpy as jnp
from jax.experimental import pallas as pl
from jax.experimental.pallas import tpu as pltpu
from jax.experimental.pallas import tpu_sc as plsc

N = 16384
D = 32
K = 5
TOPK = K + 1
BQ = 256
G = N // BQ
LW = 8
M = N * LW
GW = 128
BIGF = float(N)


def _normalize_kernel(x_ref, o_ref):
    x = x_ref[...]
    n = jnp.sqrt(jnp.sum(x * x, axis=1, keepdims=True))
    o_ref[...] = x / jnp.maximum(n, 1e-12)


def _knn_kernel(qn_ref, knt_ref, out_ref):
    qn = qn_ref[...]
    knt = knt_ref[...]
    sims = jax.lax.dot_general(
        qn, knt, (((1,), (0,)), ((), ())),
        preferred_element_type=jnp.float32)

    iotaf = jax.lax.broadcasted_iota(
        jnp.int32, (BQ, N), 1).astype(jnp.float32)
    l8 = jax.lax.broadcasted_iota(jnp.int32, (BQ, LW), 1)

    idxs = jnp.zeros((BQ, LW), jnp.float32)
    for k in range(TOPK):
        m = jnp.max(sims, axis=1, keepdims=True)
        key = jnp.min(jnp.where(sims == m, iotaf, BIGF), axis=1,
                      keepdims=True)
        idxs = jnp.where(l8 == k, key, idxs)
        if k < TOPK - 1:
            sims = jnp.where(iotaf == key, -jnp.inf, sims)
    out_ref[...] = idxs.astype(jnp.int32)


def _sc_gather(kid_tiled, idx_flat):
    mesh = plsc.VectorSubcoreMesh(
        core_axis_name="core", subcore_axis_name="subcore")

    @pl.kernel(out_type=jax.ShapeDtypeStruct((M, 128), jnp.int32),
               mesh=mesh)
    def kern(kid_hbm, i_hbm, o_hbm):
        def body(i_vmem, o_vmem):
            pltpu.sync_copy(kid_hbm.at[i_vmem.at[0]], o_vmem)

        pltpu.emit_pipeline(
            body,
            grid=(M // GW,),
            in_specs=[pl.BlockSpec((1, GW), index_map=lambda i: (0, i))],
            out_specs=[pl.BlockSpec((GW, 128), index_map=lambda i: (i, 0))],
            core_axis_name="subcore",
            dimension_semantics=(pltpu.PARALLEL,),
        )(i_hbm, o_hbm)

    return kern(kid_tiled, idx_flat)


def kernel(query_ids, query_embed, key_ids, key_embed):
    norm = pl.pallas_call(
        _normalize_kernel,
        grid=(G,),
        in_specs=[pl.BlockSpec((BQ, D), lambda i: (i, 0))],
        out_specs=pl.BlockSpec((BQ, D), lambda i: (i, 0)),
        out_shape=jax.ShapeDtypeStruct((N, D), jnp.float32),
        compiler_params=pltpu.CompilerParams(
            dimension_semantics=("parallel",)),
    )
    qn = norm(query_embed)
    kn = norm(key_embed)
    knt = kn.T

    top_idx = pl.pallas_call(
        _knn_kernel,
        grid=(G,),
        in_specs=[
            pl.BlockSpec((BQ, D), lambda i: (i, 0)),
            pl.BlockSpec((D, N), lambda i: (0, 0)),
        ],
        out_specs=pl.BlockSpec((BQ, LW), lambda i: (i, 0)),
        out_shape=jax.ShapeDtypeStruct((N, LW), jnp.int32),
        compiler_params=pltpu.CompilerParams(
            dimension_semantics=("parallel",)),
    )(qn, knt)

    kid_tiled = jnp.broadcast_to(key_ids[:, None], (N, 128))
    gathered = _sc_gather(kid_tiled, top_idx.reshape(1, M))

    lane = jnp.arange(LW)
    valid = (lane >= 1) & (lane <= K)
    qrep = jnp.where(valid[None, :], query_ids[:, None], -1)
    matches = (gathered[:, 0].reshape(N, LW) == qrep).astype(jnp.float32)
    return jnp.sum(matches) / jnp.float32(N * K)


# speedup vs baseline: 2.7060x; 2.7060x over previous
import jax
import jax.numpy as jnp
from jax.experimental import pallas as pl
from jax.experimental.pallas import tpu as pltpu

N = 16384
D = 32
K = 5
TOPK = K + 1
BQ = 256
G = N // BQ


def _normalize_kernel(x_ref, o_ref):
    x = x_ref[...]
    n = jnp.sqrt(jnp.sum(x * x, axis=1, keepdims=True))
    o_ref[...] = x / jnp.maximum(n, 1e-12)


def _knn_kernel(qid_ref, qn_ref, kid_ref, knt_ref, out_ref):
    qn = qn_ref[...]
    knt = knt_ref[...]
    sims = jax.lax.dot_general(
        qn, knt, (((1,), (0,)), ((), ())),
        preferred_element_type=jnp.float32)

    qid = qid_ref[...]
    kid = kid_ref[...]
    match = (qid == kid)

    iota2 = jax.lax.broadcasted_iota(jnp.int32, (BQ, N), 1) * 2 + 1
    piota = jnp.where(match, iota2 - 1, iota2).astype(jnp.float32)

    acc = jnp.zeros((BQ, 1), jnp.int32)
    for k in range(TOPK):
        m = jnp.max(sims, axis=1, keepdims=True)
        key = jnp.min(jnp.where(sims == m, piota, float(2 * N)), axis=1,
                      keepdims=True)
        if k > 0:
            acc = acc + (1 - (key.astype(jnp.int32) & 1))
        if k < TOPK - 1:
            sims = jnp.where(piota == key, -jnp.inf, sims)
    out_ref[...] = acc.astype(jnp.float32)


def kernel(query_ids, query_embed, key_ids, key_embed):
    norm = pl.pallas_call(
        _normalize_kernel,
        grid=(G,),
        in_specs=[pl.BlockSpec((BQ, D), lambda i: (i, 0))],
        out_specs=pl.BlockSpec((BQ, D), lambda i: (i, 0)),
        out_shape=jax.ShapeDtypeStruct((N, D), jnp.float32),
        compiler_params=pltpu.CompilerParams(
            dimension_semantics=("parallel",)),
    )
    qn = norm(query_embed)
    kn = norm(key_embed)
    knt = kn.T

    counts = pl.pallas_call(
        _knn_kernel,
        grid=(G,),
        in_specs=[
            pl.BlockSpec((BQ, 1), lambda i: (i, 0)),
            pl.BlockSpec((BQ, D), lambda i: (i, 0)),
            pl.BlockSpec((1, N), lambda i: (0, 0)),
            pl.BlockSpec((D, N), lambda i: (0, 0)),
        ],
        out_specs=pl.BlockSpec((BQ, 1), lambda i: (i, 0)),
        out_shape=jax.ShapeDtypeStruct((N, 1), jnp.float32),
        compiler_params=pltpu.CompilerParams(
            dimension_semantics=("parallel",)),
    )(query_ids.reshape(N, 1), qn, key_ids.reshape(1, N), knt)

    return jnp.sum(counts) / jnp.float32(N * K)
